# Initial kernel scaffold; baseline (speedup 1.0000x reference)
import jax
import jax.numpy as jnp
from jax.experimental import pallas as pl


def kernel(g, input, inputs, W1, b1, W2, b2, Wc, bc, Wa, ba, Wv, bv):
    raise NotImplementedError("write your pallas kernel here")


# speedup vs baseline: 5.2312x; 5.2312x over previous
import functools

import jax
import jax.numpy as jnp
from jax import lax
from jax.experimental import pallas as pl
from jax.experimental.pallas import tpu as pltpu
from jax.experimental.pallas import tpu_sc as plsc

_N = 10000
_E = 320000
_S = 6
_DIN = 2624
_DH = 128

_NC = 2
_NS = 16
_NW = _NC * _NS
_LANE = 128
_EROWS = _E // _LANE
_RPW = _EROWS // _NW
_EXTRA = _EROWS - _RPW * _NW

_DEGP = 2 * _N
_DEGZ = 16 * 1280

_mesh = plsc.VectorSubcoreMesh(core_axis_name="c", subcore_axis_name="s")


def _worker_rows(w):
    return _RPW + (w < _EXTRA).astype(jnp.int32)


@functools.partial(
    pl.kernel,
    out_type=jax.ShapeDtypeStruct((_NC, _DEGZ), jnp.float32),
    mesh=_mesh,
    compiler_params=pltpu.CompilerParams(use_tc_tiling_on_sc=False),
    scratch_types=[
        pltpu.VMEM((_LANE,), jnp.int32),
        pltpu.VMEM((_LANE,), jnp.float32),
        pltpu.VMEM((1280,), jnp.float32),
        pltpu.VMEM_SHARED((_DEGZ,), jnp.float32),
    ],
)
def _sc_degrees(src_hbm, dst_hbm, out_hbm, idx_v, ones_v, zero_v, deg_sh):
    c = lax.axis_index("c")
    s = lax.axis_index("s")
    w = s * _NC + c

    for j in range(_LANE // 16):
        ones_v[pl.ds(j * 16, 16)] = jnp.ones((16,), jnp.float32)

    def _zb(j, carry):
        zero_v[pl.ds(j * 16, 16)] = jnp.zeros((16,), jnp.float32)
        return carry

    lax.fori_loop(0, 1280 // 16, _zb, 0)
    pltpu.sync_copy(zero_v, deg_sh.at[pl.ds(s * 1280, 1280)])
    plsc.subcore_barrier()

    def _body(i, carry):
        r = w + i * _NW
        pltpu.sync_copy(src_hbm.at[r], idx_v)
        pltpu.sync_copy(ones_v, deg_sh.at[idx_v], add=True)
        pltpu.sync_copy(dst_hbm.at[r], idx_v)
        for j in range(_LANE // 16):
            idx_v[pl.ds(j * 16, 16)] = idx_v[pl.ds(j * 16, 16)] + _N
        pltpu.sync_copy(ones_v, deg_sh.at[idx_v], add=True)
        return carry

    lax.fori_loop(0, _worker_rows(w), _body, 0)
    plsc.subcore_barrier()
    pltpu.sync_copy(deg_sh.at[pl.ds(s * 1280, 1280)],
                    out_hbm.at[c, pl.ds(s * 1280, 1280)])


def _make_agg(d, tc_tiling):
    slab = 640
    nz = 80

    @functools.partial(
        pl.kernel,
        out_type=jax.ShapeDtypeStruct((_NC, _N, d), jnp.float32),
        mesh=_mesh,
        compiler_params=pltpu.CompilerParams(use_tc_tiling_on_sc=tc_tiling),
        scratch_types=[
            pltpu.VMEM((_LANE,), jnp.int32),
            pltpu.VMEM((_LANE,), jnp.int32),
            pltpu.VMEM((_LANE, d), jnp.float32),
            pltpu.VMEM((nz, d), jnp.float32),
            pltpu.VMEM_SHARED((_N, d), jnp.float32),
            pltpu.SemaphoreType.DMA,
        ],
    )
    def _agg(h_hbm, src_hbm, dst_hbm, out_hbm, srcv, dstv, rows_v, zv, acc_sh,
             sem):
        c = lax.axis_index("c")
        s = lax.axis_index("s")
        w = s * _NC + c

        def _zb(i, carry):
            for j in range(d // 16):
                zv[i, pl.ds(j * 16, 16)] = jnp.zeros((16,), jnp.float32)
            return carry

        lax.fori_loop(0, nz, _zb, 0)
        base = s * slab
        for k in range(400 // nz):
            pltpu.sync_copy(zv, acc_sh.at[pl.ds(base + k * nz, nz)])

        @pl.when(s < _NS - 1)
        def _():
            for k in range(400 // nz, slab // nz):
                pltpu.sync_copy(zv, acc_sh.at[pl.ds(base + k * nz, nz)])

        plsc.subcore_barrier()

        def _body(i, carry):
            r = w + i * _NW
            pltpu.sync_copy(src_hbm.at[r], srcv)
            pltpu.sync_copy(dst_hbm.at[r], dstv)
            pltpu.async_copy(h_hbm.at[srcv], rows_v, sem).wait()
            pltpu.sync_copy(rows_v, acc_sh.at[dstv], add=True)
            return carry

        lax.fori_loop(0, _worker_rows(w), _body, 0)
        plsc.subcore_barrier()
        pltpu.sync_copy(acc_sh.at[pl.ds(base, 400)],
                        out_hbm.at[c, pl.ds(base, 400)])

        @pl.when(s < _NS - 1)
        def _():
            pltpu.sync_copy(acc_sh.at[pl.ds(base + 400, 240)],
                            out_hbm.at[c, pl.ds(base + 400, 240)])

    return _agg


_agg128 = _make_agg(_DH, True)
_agg16 = _make_agg(16, False)


_BM = 400


def _mm1_body(x_ref, w_ref, y_ref, cs_ref, mm_ref, acc):
    i = pl.program_id(0)
    xb = x_ref[...]
    y_ref[...] = jnp.dot(xb, w_ref[...], preferred_element_type=jnp.float32)
    bmn = jnp.min(xb)
    bmx = jnp.max(xb)

    @pl.when(i == 0)
    def _():
        acc[0] = bmn
        acc[1] = bmx

    @pl.when(i > 0)
    def _():
        acc[0] = jnp.minimum(acc[0], bmn)
        acc[1] = jnp.maximum(acc[1], bmx)

    @pl.when(i == pl.num_programs(0) - 1)
    def _():
        mm_ref[0, 0] = acc[0]
        mm_ref[0, 1] = acc[1]

    cs_ref[...] = jnp.sum(w_ref[...], axis=0, keepdims=True)


def _mm1(x, w1):
    return pl.pallas_call(
        _mm1_body,
        grid=(_N // _BM,),
        in_specs=[
            pl.BlockSpec((_BM, _DIN), lambda i: (i, 0)),
            pl.BlockSpec((_DIN, _DH), lambda i: (0, 0)),
        ],
        out_specs=[
            pl.BlockSpec((_BM, _DH), lambda i: (i, 0)),
            pl.BlockSpec((1, _DH), lambda i: (0, 0)),
            pl.BlockSpec((1, 2), lambda i: (0, 0), memory_space=pltpu.SMEM),
        ],
        out_shape=[
            jax.ShapeDtypeStruct((_N, _DH), jnp.float32),
            jax.ShapeDtypeStruct((1, _DH), jnp.float32),
            jax.ShapeDtypeStruct((1, 2), jnp.float32),
        ],
        scratch_shapes=[pltpu.SMEM((2,), jnp.float32)],
    )(x, w1)


def _scale_body(y_ref, cs_ref, mm_ref, no_ref, h_ref):
    mn = mm_ref[0, 0]
    inv = 1.0 / (mm_ref[0, 1] - mn)
    h_ref[...] = (y_ref[...] - mn * cs_ref[...]) * inv * no_ref[...]


def _scale(y, cs, mm, no):
    return pl.pallas_call(
        _scale_body,
        grid=(_N // _BM,),
        in_specs=[
            pl.BlockSpec((_BM, _DH), lambda i: (i, 0)),
            pl.BlockSpec((1, _DH), lambda i: (0, 0)),
            pl.BlockSpec((1, 2), lambda i: (0, 0), memory_space=pltpu.SMEM),
            pl.BlockSpec((_BM, 1), lambda i: (i, 0)),
        ],
        out_specs=pl.BlockSpec((_BM, _DH), lambda i: (i, 0)),
        out_shape=jax.ShapeDtypeStruct((_N, _DH), jnp.float32),
    )(y, cs, mm, no)


def _mid_body(a_ref, ni_ref, b1_ref, w2_ref, no_ref, h2_ref):
    x1 = jax.nn.relu((a_ref[0] + a_ref[1]) * ni_ref[...] + b1_ref[...])
    h2_ref[...] = jnp.dot(
        x1, w2_ref[...], preferred_element_type=jnp.float32) * no_ref[...]


def _mid(agg1, ni, b1, w2p, no):
    return pl.pallas_call(
        _mid_body,
        grid=(_N // _BM,),
        in_specs=[
            pl.BlockSpec((_NC, _BM, _DH), lambda i: (0, i, 0)),
            pl.BlockSpec((_BM, 1), lambda i: (i, 0)),
            pl.BlockSpec((1, _DH), lambda i: (0, 0)),
            pl.BlockSpec((_DH, 16), lambda i: (0, 0)),
            pl.BlockSpec((_BM, 1), lambda i: (i, 0)),
        ],
        out_specs=pl.BlockSpec((_BM, 16), lambda i: (i, 0)),
        out_shape=jax.ShapeDtypeStruct((_N, 16), jnp.float32),
    )(agg1, ni, b1, w2p, no)


def _post2_body(a_ref, ni_ref, b2_ref, x2_ref):
    v = (a_ref[0] + a_ref[1]) * ni_ref[...]
    x2_ref[...] = v[:, :_S] + b2_ref[...]


def _post2(agg2, ni, b2):
    return pl.pallas_call(
        _post2_body,
        grid=(_N // _BM,),
        in_specs=[
            pl.BlockSpec((_NC, _BM, 16), lambda i: (0, i, 0)),
            pl.BlockSpec((_BM, 1), lambda i: (i, 0)),
            pl.BlockSpec((1, _S), lambda i: (0, 0)),
        ],
        out_specs=pl.BlockSpec((_BM, _S), lambda i: (i, 0)),
        out_shape=jax.ShapeDtypeStruct((_N, _S), jnp.float32),
    )(agg2, ni, b2)


_ZTOT = _N * _S + _N + _S + 1
_BK = 3584
_KSTEPS = 20
_ZPAD = _KSTEPS * _BK


def _final_body(z_ref, wc_ref, bc_ref, wa_ref, ba_ref, wv_ref, bv_ref,
                a_ref, c_ref, acc):
    k = pl.program_id(0)
    rows = lax.broadcasted_iota(jnp.int32, (_BK, _DH), 0) + k * _BK
    wc = jnp.where(rows < _ZTOT, wc_ref[...], 0.0)
    part = jnp.dot(z_ref[0], wc, preferred_element_type=jnp.float32)

    @pl.when(k == 0)
    def _():
        acc[...] = part

    @pl.when(k > 0)
    def _():
        acc[...] = acc[...] + part

    @pl.when(k == _KSTEPS - 1)
    def _():
        h = jax.nn.relu(acc[...] + bc_ref[...])
        a_ref[...] = jnp.dot(
            h, wa_ref[...], preferred_element_type=jnp.float32) + ba_ref[...]
        c_ref[...] = jnp.dot(
            h, wv_ref[...], preferred_element_type=jnp.float32) + bv_ref[...]


def _final(z3, wc, bc, wa, ba, wv, bv):
    return pl.pallas_call(
        _final_body,
        grid=(_KSTEPS,),
        in_specs=[
            pl.BlockSpec((1, 1, _BK), lambda k: (k, 0, 0)),
            pl.BlockSpec((_BK, _DH), lambda k: (k, 0)),
            pl.BlockSpec((1, _DH), lambda k: (0, 0)),
            pl.BlockSpec((_DH, _S), lambda k: (0, 0)),
            pl.BlockSpec((1, _S), lambda k: (0, 0)),
            pl.BlockSpec((_DH, 1), lambda k: (0, 0)),
            pl.BlockSpec((1, 1), lambda k: (0, 0)),
        ],
        out_specs=[
            pl.BlockSpec((1, _S), lambda k: (0, 0)),
            pl.BlockSpec((1, 1), lambda k: (0, 0)),
        ],
        out_shape=[
            jax.ShapeDtypeStruct((1, _S), jnp.float32),
            jax.ShapeDtypeStruct((1, 1), jnp.float32),
        ],
        scratch_shapes=[pltpu.VMEM((1, _DH), jnp.float32)],
    )(z3, wc, bc, wa, ba, wv, bv)


def kernel(g, input, inputs, W1, b1, W2, b2, Wc, bc, Wa, ba, Wv, bv):
    src2 = g[0].reshape(_EROWS, _LANE)
    dst2 = g[1].reshape(_EROWS, _LANE)

    degp = _sc_degrees(src2, dst2)
    degs = degp[0, :_DEGP] + degp[1, :_DEGP]
    no = (jnp.clip(degs[:_N], 1.0, None) ** -0.5).reshape(_N, 1)
    ni = (jnp.clip(degs[_N:], 1.0, None) ** -0.5).reshape(_N, 1)

    y, cs, mm = _mm1(input, W1)
    h1 = _scale(y, cs, mm, no)
    agg1 = _agg128(h1, src2, dst2)

    w2p = jnp.zeros((_DH, 16), jnp.float32).at[:, :_S].set(W2)
    h2 = _mid(agg1, ni, b1.reshape(1, _DH), w2p, no)
    agg2 = _agg16(h2, src2, dst2)
    x2 = _post2(agg2, ni, b2.reshape(1, _S))

    zp = jnp.concatenate(
        [inputs, x2.reshape(-1),
         jnp.zeros((_ZPAD - _ZTOT,), jnp.float32)])
    z3 = zp.reshape(_KSTEPS, 1, _BK)
    actor, critic = _final(z3, Wc, bc.reshape(1, _DH), Wa,
                           ba.reshape(1, _S), Wv, bv.reshape(1, 1))
    return (actor.reshape(-1), critic.reshape(-1))


# speedup vs baseline: 8.0758x; 1.5438x over previous
import functools

import jax
import jax.numpy as jnp
from jax import lax
from jax.experimental import pallas as pl
from jax.experimental.pallas import tpu as pltpu
from jax.experimental.pallas import tpu_sc as plsc

_N = 10000
_E = 320000
_S = 6
_DIN = 2624
_DH = 128

_NC = 2
_NS = 16
_NW = _NC * _NS
_LANE = 128
_EROWS = _E // _LANE
_RPW = _EROWS // _NW
_EXTRA = _EROWS - _RPW * _NW

_mesh = plsc.VectorSubcoreMesh(core_axis_name="c", subcore_axis_name="s")


_K = 3
_SLOTS = _RPW // _K
_NP = 10240


def _row0(w):
    return w * _RPW + jnp.minimum(w, _EXTRA)


@functools.partial(
    pl.kernel,
    out_type=jax.ShapeDtypeStruct((_NC, 2 * _NP), jnp.float32),
    mesh=_mesh,
    compiler_params=pltpu.CompilerParams(use_tc_tiling_on_sc=False),
    scratch_types=[
        pltpu.VMEM((2, _K, _LANE), jnp.int32),
        pltpu.VMEM((2, _K, _LANE), jnp.int32),
        pltpu.VMEM((_LANE,), jnp.float32),
        pltpu.VMEM((1280,), jnp.float32),
        pltpu.VMEM_SHARED((2 * _NP,), jnp.float32),
        pltpu.SemaphoreType.DMA,
        pltpu.SemaphoreType.DMA,
    ],
)
def _sc_degrees(src_hbm, dst_hbm, out_hbm, srcv, dstv, ones_v, zero_v,
                deg_sh, sem_i0, sem_i1):
    c = lax.axis_index("c")
    s = lax.axis_index("s")
    w = s * _NC + c
    row0 = _row0(w)
    sems = (sem_i0, sem_i1)

    for j in range(_LANE // 16):
        ones_v[pl.ds(j * 16, 16)] = jnp.ones((16,), jnp.float32)

    def _zb(j, carry):
        zero_v[pl.ds(j * 16, 16)] = jnp.zeros((16,), jnp.float32)
        return carry

    lax.fori_loop(0, 1280 // 16, _zb, 0)
    pltpu.sync_copy(zero_v, deg_sh.at[pl.ds(s * 1280, 1280)])
    plsc.subcore_barrier()

    def _adjust(idxv, b):
        for j in range(_K):
            for jj in range(_LANE // 16):
                idxv[b, j, pl.ds(jj * 16, 16)] = (
                    idxv[b, j, pl.ds(jj * 16, 16)] + _NP)

    def _fetch(t, b):
        r = row0 + t * _K
        pltpu.async_copy(src_hbm.at[pl.ds(r, _K)], srcv.at[b], sems[b])
        pltpu.async_copy(dst_hbm.at[pl.ds(r, _K)], dstv.at[b], sems[b])

    def _drain(b):
        pltpu.make_async_copy(src_hbm.at[pl.ds(0, _K)], srcv.at[b],
                              sems[b]).wait()
        pltpu.make_async_copy(dst_hbm.at[pl.ds(0, _K)], dstv.at[b],
                              sems[b]).wait()
        _adjust(dstv, b)
        for j in range(_K):
            pltpu.sync_copy(ones_v, deg_sh.at[srcv.at[b, j]], add=True)
            pltpu.sync_copy(ones_v, deg_sh.at[dstv.at[b, j]], add=True)

    _fetch(0, 0)

    def _pair(u, carry):
        t0 = 2 * u
        _fetch(t0 + 1, 1)
        _drain(0)

        @pl.when(t0 + 2 < _SLOTS)
        def _():
            _fetch(t0 + 2, 0)

        _drain(1)
        return carry

    lax.fori_loop(0, _SLOTS // 2, _pair, 0)

    @pl.when(w < _EXTRA)
    def _():
        r = row0 + _RPW
        pltpu.sync_copy(src_hbm.at[pl.ds(r, 1)], srcv.at[0, pl.ds(0, 1)])
        pltpu.sync_copy(dst_hbm.at[pl.ds(r, 1)], dstv.at[0, pl.ds(0, 1)])
        for jj in range(_LANE // 16):
            dstv[0, 0, pl.ds(jj * 16, 16)] = (
                dstv[0, 0, pl.ds(jj * 16, 16)] + _NP)
        pltpu.sync_copy(ones_v, deg_sh.at[srcv.at[0, 0]], add=True)
        pltpu.sync_copy(ones_v, deg_sh.at[dstv.at[0, 0]], add=True)

    plsc.subcore_barrier()
    pltpu.sync_copy(deg_sh.at[pl.ds(s * 1280, 1280)],
                    out_hbm.at[c, pl.ds(s * 1280, 1280)])


def _make_agg(d, tc_tiling, k):
    slab = 640
    nz = 40
    slots = _RPW // k
    assert slots % 2 == 0 and slots * k == _RPW

    @functools.partial(
        pl.kernel,
        out_type=jax.ShapeDtypeStruct((_NC, _N, d), jnp.float32),
        mesh=_mesh,
        compiler_params=pltpu.CompilerParams(use_tc_tiling_on_sc=tc_tiling),
        scratch_types=[
            pltpu.VMEM((2, k, _LANE), jnp.int32),
            pltpu.VMEM((2, k, _LANE), jnp.int32),
            pltpu.VMEM((2, k * _LANE, d), jnp.float32),
            pltpu.VMEM((nz, d), jnp.float32),
            pltpu.VMEM_SHARED((_N, d), jnp.float32),
            pltpu.SemaphoreType.DMA,
            pltpu.SemaphoreType.DMA,
            pltpu.SemaphoreType.DMA,
            pltpu.SemaphoreType.DMA,
        ],
    )
    def _agg(h_hbm, src_hbm, dst_hbm, out_hbm, srcv, dstv, rows_v, zv, acc_sh,
             sem_i0, sem_i1, sem_g0, sem_g1):
        c = lax.axis_index("c")
        s = lax.axis_index("s")
        w = s * _NC + c
        row0 = _row0(w)
        sems_i = (sem_i0, sem_i1)
        sems_g = (sem_g0, sem_g1)

        def _zb(i, carry):
            for j in range(d // 16):
                zv[i, pl.ds(j * 16, 16)] = jnp.zeros((16,), jnp.float32)
            return carry

        lax.fori_loop(0, nz, _zb, 0)
        base = s * slab
        for q in range(400 // nz):
            pltpu.sync_copy(zv, acc_sh.at[pl.ds(base + q * nz, nz)])

        @pl.when(s < _NS - 1)
        def _():
            for q in range(400 // nz, slab // nz):
                pltpu.sync_copy(zv, acc_sh.at[pl.ds(base + q * nz, nz)])

        plsc.subcore_barrier()

        def _fetch(t, b):
            r = row0 + t * k
            for j in range(k):
                pltpu.async_copy(src_hbm.at[r + j], srcv.at[b, j], sems_i[b])
                pltpu.async_copy(dst_hbm.at[r + j], dstv.at[b, j], sems_i[b])
            for j in range(k):
                pltpu.make_async_copy(src_hbm.at[0], srcv.at[b, j],
                                      sems_i[b]).wait()
                pltpu.make_async_copy(dst_hbm.at[0], dstv.at[b, j],
                                      sems_i[b]).wait()
            for j in range(k):
                pltpu.async_copy(h_hbm.at[srcv.at[b, j]],
                                 rows_v.at[b, pl.ds(j * _LANE, _LANE)],
                                 sems_g[b])

        def _drain(b):
            for j in range(k):
                pltpu.make_async_copy(h_hbm.at[pl.ds(0, _LANE)],
                                      rows_v.at[b, pl.ds(j * _LANE, _LANE)],
                                      sems_g[b]).wait()
            for j in range(k):
                pltpu.sync_copy(rows_v.at[b, pl.ds(j * _LANE, _LANE)],
                                acc_sh.at[dstv.at[b, j]], add=True)

        _fetch(0, 0)

        def _pair(u, carry):
            t0 = 2 * u
            _fetch(t0 + 1, 1)
            _drain(0)

            @pl.when(t0 + 2 < slots)
            def _():
                _fetch(t0 + 2, 0)

            _drain(1)
            return carry

        lax.fori_loop(0, slots // 2, _pair, 0)

        @pl.when(w < _EXTRA)
        def _():
            r = row0 + _RPW
            pltpu.sync_copy(src_hbm.at[r], srcv.at[0, 0])
            pltpu.sync_copy(dst_hbm.at[r], dstv.at[0, 0])
            pltpu.async_copy(h_hbm.at[srcv.at[0, 0]],
                             rows_v.at[0, pl.ds(0, _LANE)], sem_g0)
            pltpu.make_async_copy(h_hbm.at[pl.ds(0, _LANE)],
                                  rows_v.at[0, pl.ds(0, _LANE)], sem_g0).wait()
            pltpu.sync_copy(rows_v.at[0, pl.ds(0, _LANE)],
                            acc_sh.at[dstv.at[0, 0]], add=True)

        plsc.subcore_barrier()
        pltpu.sync_copy(acc_sh.at[pl.ds(base, 400)],
                        out_hbm.at[c, pl.ds(base, 400)])

        @pl.when(s < _NS - 1)
        def _():
            pltpu.sync_copy(acc_sh.at[pl.ds(base + 400, 240)],
                            out_hbm.at[c, pl.ds(base + 400, 240)])

    return _agg


_agg128 = _make_agg(_DH, True, 1)
_agg16 = _make_agg(16, False, 3)


_BM = 400


def _mm1_body(x_ref, w_ref, y_ref, cs_ref, mm_ref, acc):
    i = pl.program_id(0)
    xb = x_ref[...]
    y_ref[...] = jnp.dot(xb, w_ref[...], preferred_element_type=jnp.float32)
    bmn = jnp.min(xb)
    bmx = jnp.max(xb)

    @pl.when(i == 0)
    def _():
        acc[0] = bmn
        acc[1] = bmx

    @pl.when(i > 0)
    def _():
        acc[0] = jnp.minimum(acc[0], bmn)
        acc[1] = jnp.maximum(acc[1], bmx)

    @pl.when(i == pl.num_programs(0) - 1)
    def _():
        mm_ref[0, 0] = acc[0]
        mm_ref[0, 1] = acc[1]

    cs_ref[...] = jnp.sum(w_ref[...], axis=0, keepdims=True)


def _mm1(x, w1):
    return pl.pallas_call(
        _mm1_body,
        grid=(_N // _BM,),
        in_specs=[
            pl.BlockSpec((_BM, _DIN), lambda i: (i, 0)),
            pl.BlockSpec((_DIN, _DH), lambda i: (0, 0)),
        ],
        out_specs=[
            pl.BlockSpec((_BM, _DH), lambda i: (i, 0)),
            pl.BlockSpec((1, _DH), lambda i: (0, 0)),
            pl.BlockSpec((1, 2), lambda i: (0, 0), memory_space=pltpu.SMEM),
        ],
        out_shape=[
            jax.ShapeDtypeStruct((_N, _DH), jnp.float32),
            jax.ShapeDtypeStruct((1, _DH), jnp.float32),
            jax.ShapeDtypeStruct((1, 2), jnp.float32),
        ],
        scratch_shapes=[pltpu.SMEM((2,), jnp.float32)],
    )(x, w1)


def _scale_body(y_ref, cs_ref, mm_ref, no_ref, h_ref):
    mn = mm_ref[0, 0]
    inv = 1.0 / (mm_ref[0, 1] - mn)
    h_ref[...] = (y_ref[...] - mn * cs_ref[...]) * inv * no_ref[...]


def _scale(y, cs, mm, no):
    return pl.pallas_call(
        _scale_body,
        grid=(_N // _BM,),
        in_specs=[
            pl.BlockSpec((_BM, _DH), lambda i: (i, 0)),
            pl.BlockSpec((1, _DH), lambda i: (0, 0)),
            pl.BlockSpec((1, 2), lambda i: (0, 0), memory_space=pltpu.SMEM),
            pl.BlockSpec((_BM, 1), lambda i: (i, 0)),
        ],
        out_specs=pl.BlockSpec((_BM, _DH), lambda i: (i, 0)),
        out_shape=jax.ShapeDtypeStruct((_N, _DH), jnp.float32),
    )(y, cs, mm, no)


def _mid_body(a_ref, ni_ref, b1_ref, w2_ref, no_ref, h2_ref):
    x1 = jax.nn.relu((a_ref[0] + a_ref[1]) * ni_ref[...] + b1_ref[...])
    h2_ref[...] = jnp.dot(
        x1, w2_ref[...], preferred_element_type=jnp.float32) * no_ref[...]


def _mid(agg1, ni, b1, w2p, no):
    return pl.pallas_call(
        _mid_body,
        grid=(_N // _BM,),
        in_specs=[
            pl.BlockSpec((_NC, _BM, _DH), lambda i: (0, i, 0)),
            pl.BlockSpec((_BM, 1), lambda i: (i, 0)),
            pl.BlockSpec((1, _DH), lambda i: (0, 0)),
            pl.BlockSpec((_DH, 16), lambda i: (0, 0)),
            pl.BlockSpec((_BM, 1), lambda i: (i, 0)),
        ],
        out_specs=pl.BlockSpec((_BM, 16), lambda i: (i, 0)),
        out_shape=jax.ShapeDtypeStruct((_N, 16), jnp.float32),
    )(agg1, ni, b1, w2p, no)


def _post2_body(a_ref, ni_ref, b2_ref, x2_ref):
    v = (a_ref[0] + a_ref[1]) * ni_ref[...]
    x2_ref[...] = v[:, :_S] + b2_ref[...]


def _post2(agg2, ni, b2):
    return pl.pallas_call(
        _post2_body,
        grid=(_N // _BM,),
        in_specs=[
            pl.BlockSpec((_NC, _BM, 16), lambda i: (0, i, 0)),
            pl.BlockSpec((_BM, 1), lambda i: (i, 0)),
            pl.BlockSpec((1, _S), lambda i: (0, 0)),
        ],
        out_specs=pl.BlockSpec((_BM, _S), lambda i: (i, 0)),
        out_shape=jax.ShapeDtypeStruct((_N, _S), jnp.float32),
    )(agg2, ni, b2)


_ZTOT = _N * _S + _N + _S + 1
_BK = 3584
_KSTEPS = 20
_ZPAD = _KSTEPS * _BK


def _final_body(z_ref, wc_ref, bc_ref, wa_ref, ba_ref, wv_ref, bv_ref,
                a_ref, c_ref, acc):
    k = pl.program_id(0)
    rows = lax.broadcasted_iota(jnp.int32, (_BK, _DH), 0) + k * _BK
    wc = jnp.where(rows < _ZTOT, wc_ref[...], 0.0)
    part = jnp.dot(z_ref[0], wc, preferred_element_type=jnp.float32)

    @pl.when(k == 0)
    def _():
        acc[...] = part

    @pl.when(k > 0)
    def _():
        acc[...] = acc[...] + part

    @pl.when(k == _KSTEPS - 1)
    def _():
        h = jax.nn.relu(acc[...] + bc_ref[...])
        a_ref[...] = jnp.dot(
            h, wa_ref[...], preferred_element_type=jnp.float32) + ba_ref[...]
        c_ref[...] = jnp.dot(
            h, wv_ref[...], preferred_element_type=jnp.float32) + bv_ref[...]


def _final(z3, wc, bc, wa, ba, wv, bv):
    return pl.pallas_call(
        _final_body,
        grid=(_KSTEPS,),
        in_specs=[
            pl.BlockSpec((1, 1, _BK), lambda k: (k, 0, 0)),
            pl.BlockSpec((_BK, _DH), lambda k: (k, 0)),
            pl.BlockSpec((1, _DH), lambda k: (0, 0)),
            pl.BlockSpec((_DH, _S), lambda k: (0, 0)),
            pl.BlockSpec((1, _S), lambda k: (0, 0)),
            pl.BlockSpec((_DH, 1), lambda k: (0, 0)),
            pl.BlockSpec((1, 1), lambda k: (0, 0)),
        ],
        out_specs=[
            pl.BlockSpec((1, _S), lambda k: (0, 0)),
            pl.BlockSpec((1, 1), lambda k: (0, 0)),
        ],
        out_shape=[
            jax.ShapeDtypeStruct((1, _S), jnp.float32),
            jax.ShapeDtypeStruct((1, 1), jnp.float32),
        ],
        scratch_shapes=[pltpu.VMEM((1, _DH), jnp.float32)],
    )(z3, wc, bc, wa, ba, wv, bv)


def kernel(g, input, inputs, W1, b1, W2, b2, Wc, bc, Wa, ba, Wv, bv):
    src2 = g[0].reshape(_EROWS, _LANE)
    dst2 = g[1].reshape(_EROWS, _LANE)

    degp = _sc_degrees(src2, dst2)
    degs = degp[0] + degp[1]
    no = (jnp.clip(degs[:_N], 1.0, None) ** -0.5).reshape(_N, 1)
    ni = (jnp.clip(degs[_NP:_NP + _N], 1.0, None) ** -0.5).reshape(_N, 1)

    y, cs, mm = _mm1(input, W1)
    h1 = _scale(y, cs, mm, no)
    agg1 = _agg128(h1, src2, dst2)

    w2p = jnp.zeros((_DH, 16), jnp.float32).at[:, :_S].set(W2)
    h2 = _mid(agg1, ni, b1.reshape(1, _DH), w2p, no)
    agg2 = _agg16(h2, src2, dst2)
    x2 = _post2(agg2, ni, b2.reshape(1, _S))

    zp = jnp.concatenate(
        [inputs, x2.reshape(-1),
         jnp.zeros((_ZPAD - _ZTOT,), jnp.float32)])
    z3 = zp.reshape(_KSTEPS, 1, _BK)
    actor, critic = _final(z3, Wc, bc.reshape(1, _DH), Wa,
                           ba.reshape(1, _S), Wv, bv.reshape(1, 1))
    return (actor.reshape(-1), critic.reshape(-1))


# speedup vs baseline: 10.3798x; 1.2853x over previous
import functools

import jax
import jax.numpy as jnp
from jax import lax
from jax.experimental import pallas as pl
from jax.experimental.pallas import tpu as pltpu
from jax.experimental.pallas import tpu_sc as plsc

_N = 10000
_E = 320000
_S = 6
_DIN = 2624
_DH = 128

_NC = 2
_NS = 16
_NW = _NC * _NS
_LANE = 128
_EROWS = _E // _LANE
_RPW = _EROWS // _NW
_EXTRA = _EROWS - _RPW * _NW

_mesh = plsc.VectorSubcoreMesh(core_axis_name="c", subcore_axis_name="s")


_K = 3
_SLOTS = _RPW // _K
_NP = 10240


def _row0(w):
    return w * _RPW + jnp.minimum(w, _EXTRA)


@functools.partial(
    pl.kernel,
    out_type=jax.ShapeDtypeStruct((_NC, 2 * _NP), jnp.float32),
    mesh=_mesh,
    compiler_params=pltpu.CompilerParams(use_tc_tiling_on_sc=False),
    scratch_types=[
        pltpu.VMEM((2, _K, _LANE), jnp.int32),
        pltpu.VMEM((2, _K, _LANE), jnp.int32),
        pltpu.VMEM((_LANE,), jnp.float32),
        pltpu.VMEM((1280,), jnp.float32),
        pltpu.VMEM_SHARED((2 * _NP,), jnp.float32),
        pltpu.SemaphoreType.DMA,
        pltpu.SemaphoreType.DMA,
    ],
)
def _sc_degrees(src_hbm, dst_hbm, out_hbm, srcv, dstv, ones_v, zero_v,
                deg_sh, sem_i0, sem_i1):
    c = lax.axis_index("c")
    s = lax.axis_index("s")
    w = s * _NC + c
    row0 = _row0(w)
    sems = (sem_i0, sem_i1)

    for j in range(_LANE // 16):
        ones_v[pl.ds(j * 16, 16)] = jnp.ones((16,), jnp.float32)

    def _zb(j, carry):
        zero_v[pl.ds(j * 16, 16)] = jnp.zeros((16,), jnp.float32)
        return carry

    lax.fori_loop(0, 1280 // 16, _zb, 0)
    pltpu.sync_copy(zero_v, deg_sh.at[pl.ds(s * 1280, 1280)])
    plsc.subcore_barrier()

    def _adjust(idxv, b):
        for j in range(_K):
            for jj in range(_LANE // 16):
                idxv[b, j, pl.ds(jj * 16, 16)] = (
                    idxv[b, j, pl.ds(jj * 16, 16)] + _NP)

    def _fetch(t, b):
        r = row0 + t * _K
        pltpu.async_copy(src_hbm.at[pl.ds(r, _K)], srcv.at[b], sems[b])
        pltpu.async_copy(dst_hbm.at[pl.ds(r, _K)], dstv.at[b], sems[b])

    def _drain(b):
        pltpu.make_async_copy(src_hbm.at[pl.ds(0, _K)], srcv.at[b],
                              sems[b]).wait()
        pltpu.make_async_copy(dst_hbm.at[pl.ds(0, _K)], dstv.at[b],
                              sems[b]).wait()
        _adjust(dstv, b)
        for j in range(_K):
            pltpu.sync_copy(ones_v, deg_sh.at[srcv.at[b, j]], add=True)
            pltpu.sync_copy(ones_v, deg_sh.at[dstv.at[b, j]], add=True)

    _fetch(0, 0)

    def _pair(u, carry):
        t0 = 2 * u
        _fetch(t0 + 1, 1)
        _drain(0)

        @pl.when(t0 + 2 < _SLOTS)
        def _():
            _fetch(t0 + 2, 0)

        _drain(1)
        return carry

    lax.fori_loop(0, _SLOTS // 2, _pair, 0)

    @pl.when(w < _EXTRA)
    def _():
        r = row0 + _RPW
        pltpu.sync_copy(src_hbm.at[pl.ds(r, 1)], srcv.at[0, pl.ds(0, 1)])
        pltpu.sync_copy(dst_hbm.at[pl.ds(r, 1)], dstv.at[0, pl.ds(0, 1)])
        for jj in range(_LANE // 16):
            dstv[0, 0, pl.ds(jj * 16, 16)] = (
                dstv[0, 0, pl.ds(jj * 16, 16)] + _NP)
        pltpu.sync_copy(ones_v, deg_sh.at[srcv.at[0, 0]], add=True)
        pltpu.sync_copy(ones_v, deg_sh.at[dstv.at[0, 0]], add=True)

    plsc.subcore_barrier()
    pltpu.sync_copy(deg_sh.at[pl.ds(s * 1280, 1280)],
                    out_hbm.at[c, pl.ds(s * 1280, 1280)])


def _make_agg(d, tc_tiling, k):
    slab = 640
    nz = 40
    slots = _RPW // k
    assert slots % 2 == 0 and slots * k == _RPW

    @functools.partial(
        pl.kernel,
        out_type=jax.ShapeDtypeStruct((_NC, _N, d), jnp.float32),
        mesh=_mesh,
        compiler_params=pltpu.CompilerParams(use_tc_tiling_on_sc=tc_tiling),
        scratch_types=[
            pltpu.VMEM((2, k, _LANE), jnp.int32),
            pltpu.VMEM((2, k, _LANE), jnp.int32),
            pltpu.VMEM((2, k * _LANE, d), jnp.float32),
            pltpu.VMEM((nz, d), jnp.float32),
            pltpu.VMEM_SHARED((_N, d), jnp.float32),
            pltpu.SemaphoreType.DMA,
            pltpu.SemaphoreType.DMA,
            pltpu.SemaphoreType.DMA,
            pltpu.SemaphoreType.DMA,
        ],
    )
    def _agg(h_hbm, src_hbm, dst_hbm, out_hbm, srcv, dstv, rows_v, zv, acc_sh,
             sem_i0, sem_i1, sem_g0, sem_g1):
        c = lax.axis_index("c")
        s = lax.axis_index("s")
        w = s * _NC + c
        row0 = _row0(w)
        sems_i = (sem_i0, sem_i1)
        sems_g = (sem_g0, sem_g1)

        def _zb(i, carry):
            for j in range(d // 16):
                zv[i, pl.ds(j * 16, 16)] = jnp.zeros((16,), jnp.float32)
            return carry

        lax.fori_loop(0, nz, _zb, 0)
        base = s * slab
        for q in range(400 // nz):
            pltpu.sync_copy(zv, acc_sh.at[pl.ds(base + q * nz, nz)])

        @pl.when(s < _NS - 1)
        def _():
            for q in range(400 // nz, slab // nz):
                pltpu.sync_copy(zv, acc_sh.at[pl.ds(base + q * nz, nz)])

        plsc.subcore_barrier()

        def _fetch(t, b):
            r = row0 + t * k
            for j in range(k):
                pltpu.async_copy(src_hbm.at[r + j], srcv.at[b, j], sems_i[b])
                pltpu.async_copy(dst_hbm.at[r + j], dstv.at[b, j], sems_i[b])
            for j in range(k):
                pltpu.make_async_copy(src_hbm.at[0], srcv.at[b, j],
                                      sems_i[b]).wait()
                pltpu.make_async_copy(dst_hbm.at[0], dstv.at[b, j],
                                      sems_i[b]).wait()
            for j in range(k):
                pltpu.async_copy(h_hbm.at[srcv.at[b, j]],
                                 rows_v.at[b, pl.ds(j * _LANE, _LANE)],
                                 sems_g[b])

        def _drain(b):
            for j in range(k):
                pltpu.make_async_copy(h_hbm.at[pl.ds(0, _LANE)],
                                      rows_v.at[b, pl.ds(j * _LANE, _LANE)],
                                      sems_g[b]).wait()
            for j in range(k):
                pltpu.sync_copy(rows_v.at[b, pl.ds(j * _LANE, _LANE)],
                                acc_sh.at[dstv.at[b, j]], add=True)

        _fetch(0, 0)

        def _pair(u, carry):
            t0 = 2 * u
            _fetch(t0 + 1, 1)
            _drain(0)

            @pl.when(t0 + 2 < slots)
            def _():
                _fetch(t0 + 2, 0)

            _drain(1)
            return carry

        lax.fori_loop(0, slots // 2, _pair, 0)

        @pl.when(w < _EXTRA)
        def _():
            r = row0 + _RPW
            pltpu.sync_copy(src_hbm.at[r], srcv.at[0, 0])
            pltpu.sync_copy(dst_hbm.at[r], dstv.at[0, 0])
            pltpu.async_copy(h_hbm.at[srcv.at[0, 0]],
                             rows_v.at[0, pl.ds(0, _LANE)], sem_g0)
            pltpu.make_async_copy(h_hbm.at[pl.ds(0, _LANE)],
                                  rows_v.at[0, pl.ds(0, _LANE)], sem_g0).wait()
            pltpu.sync_copy(rows_v.at[0, pl.ds(0, _LANE)],
                            acc_sh.at[dstv.at[0, 0]], add=True)

        plsc.subcore_barrier()
        pltpu.sync_copy(acc_sh.at[pl.ds(base, 400)],
                        out_hbm.at[c, pl.ds(base, 400)])

        @pl.when(s < _NS - 1)
        def _():
            pltpu.sync_copy(acc_sh.at[pl.ds(base + 400, 240)],
                            out_hbm.at[c, pl.ds(base + 400, 240)])

    return _agg


_agg128 = _make_agg(_DH, True, 1)
_agg16 = _make_agg(16, False, 3)


_BM = 400
_BMX = 512
_MMG = 20


def _mm1_body(x_ref, w_ref, y_ref, cs_ref, mm_ref, acc):
    i = pl.program_id(0)
    xb = x_ref[...]
    y_ref[...] = lax.dot_general(
        xb, w_ref[...], dimension_numbers=(((0,), (0,)), ((), ())),
        preferred_element_type=jnp.float32)
    col = lax.broadcasted_iota(jnp.int32, (_DIN, _BMX), 1) + i * _BMX
    valid = col < _N
    bmn = jnp.min(jnp.where(valid, xb, jnp.inf))
    bmx = jnp.max(jnp.where(valid, xb, -jnp.inf))

    @pl.when(i == 0)
    def _():
        acc[0] = bmn
        acc[1] = bmx

    @pl.when(i > 0)
    def _():
        acc[0] = jnp.minimum(acc[0], bmn)
        acc[1] = jnp.maximum(acc[1], bmx)

    @pl.when(i == pl.num_programs(0) - 1)
    def _():
        mm_ref[0, 0] = acc[0]
        mm_ref[0, 1] = acc[1]

    cs_ref[...] = jnp.sum(w_ref[...], axis=0, keepdims=True)


def _mm1(xt, w1):
    return pl.pallas_call(
        _mm1_body,
        grid=(_MMG,),
        in_specs=[
            pl.BlockSpec((_DIN, _BMX), lambda i: (0, i)),
            pl.BlockSpec((_DIN, _DH), lambda i: (0, 0)),
        ],
        out_specs=[
            pl.BlockSpec((_BMX, _DH), lambda i: (i, 0)),
            pl.BlockSpec((1, _DH), lambda i: (0, 0)),
            pl.BlockSpec((1, 2), lambda i: (0, 0), memory_space=pltpu.SMEM),
        ],
        out_shape=[
            jax.ShapeDtypeStruct((_N, _DH), jnp.float32),
            jax.ShapeDtypeStruct((1, _DH), jnp.float32),
            jax.ShapeDtypeStruct((1, 2), jnp.float32),
        ],
        scratch_shapes=[pltpu.SMEM((2,), jnp.float32)],
    )(xt, w1)


def _scale_body(y_ref, cs_ref, mm_ref, no_ref, h_ref):
    mn = mm_ref[0, 0]
    inv = 1.0 / (mm_ref[0, 1] - mn)
    h_ref[...] = (y_ref[...] - mn * cs_ref[...]) * inv * no_ref[...]


def _scale(y, cs, mm, no):
    return pl.pallas_call(
        _scale_body,
        grid=(_N // _BM,),
        in_specs=[
            pl.BlockSpec((_BM, _DH), lambda i: (i, 0)),
            pl.BlockSpec((1, _DH), lambda i: (0, 0)),
            pl.BlockSpec((1, 2), lambda i: (0, 0), memory_space=pltpu.SMEM),
            pl.BlockSpec((_BM, 1), lambda i: (i, 0)),
        ],
        out_specs=pl.BlockSpec((_BM, _DH), lambda i: (i, 0)),
        out_shape=jax.ShapeDtypeStruct((_N, _DH), jnp.float32),
    )(y, cs, mm, no)


def _mid_body(a_ref, ni_ref, b1_ref, w2_ref, no_ref, h2_ref):
    x1 = jax.nn.relu((a_ref[0] + a_ref[1]) * ni_ref[...] + b1_ref[...])
    h2_ref[...] = jnp.dot(
        x1, w2_ref[...], preferred_element_type=jnp.float32) * no_ref[...]


def _mid(agg1, ni, b1, w2p, no):
    return pl.pallas_call(
        _mid_body,
        grid=(_N // _BM,),
        in_specs=[
            pl.BlockSpec((_NC, _BM, _DH), lambda i: (0, i, 0)),
            pl.BlockSpec((_BM, 1), lambda i: (i, 0)),
            pl.BlockSpec((1, _DH), lambda i: (0, 0)),
            pl.BlockSpec((_DH, 16), lambda i: (0, 0)),
            pl.BlockSpec((_BM, 1), lambda i: (i, 0)),
        ],
        out_specs=pl.BlockSpec((_BM, 16), lambda i: (i, 0)),
        out_shape=jax.ShapeDtypeStruct((_N, 16), jnp.float32),
    )(agg1, ni, b1, w2p, no)


def _post2_body(a_ref, ni_ref, b2_ref, x2_ref):
    v = (a_ref[0] + a_ref[1]) * ni_ref[...]
    x2_ref[...] = v[:, :_S] + b2_ref[...]


def _post2(agg2, ni, b2):
    return pl.pallas_call(
        _post2_body,
        grid=(_N // _BM,),
        in_specs=[
            pl.BlockSpec((_NC, _BM, 16), lambda i: (0, i, 0)),
            pl.BlockSpec((_BM, 1), lambda i: (i, 0)),
            pl.BlockSpec((1, _S), lambda i: (0, 0)),
        ],
        out_specs=pl.BlockSpec((_BM, _S), lambda i: (i, 0)),
        out_shape=jax.ShapeDtypeStruct((_N, _S), jnp.float32),
    )(agg2, ni, b2)


_ZTOT = _N * _S + _N + _S + 1
_BK = 3584
_KSTEPS = 20
_ZPAD = _KSTEPS * _BK


def _final_body(z_ref, wc_ref, bc_ref, wa_ref, ba_ref, wv_ref, bv_ref,
                a_ref, c_ref, acc):
    k = pl.program_id(0)
    rows = lax.broadcasted_iota(jnp.int32, (_BK, _DH), 0) + k * _BK
    wc = jnp.where(rows < _ZTOT, wc_ref[...], 0.0)
    part = jnp.dot(z_ref[0], wc, preferred_element_type=jnp.float32)

    @pl.when(k == 0)
    def _():
        acc[...] = part

    @pl.when(k > 0)
    def _():
        acc[...] = acc[...] + part

    @pl.when(k == _KSTEPS - 1)
    def _():
        h = jax.nn.relu(acc[...] + bc_ref[...])
        a_ref[...] = jnp.dot(
            h, wa_ref[...], preferred_element_type=jnp.float32) + ba_ref[...]
        c_ref[...] = jnp.dot(
            h, wv_ref[...], preferred_element_type=jnp.float32) + bv_ref[...]


def _final(z3, wc, bc, wa, ba, wv, bv):
    return pl.pallas_call(
        _final_body,
        grid=(_KSTEPS,),
        in_specs=[
            pl.BlockSpec((1, 1, _BK), lambda k: (k, 0, 0)),
            pl.BlockSpec((_BK, _DH), lambda k: (k, 0)),
            pl.BlockSpec((1, _DH), lambda k: (0, 0)),
            pl.BlockSpec((_DH, _S), lambda k: (0, 0)),
            pl.BlockSpec((1, _S), lambda k: (0, 0)),
            pl.BlockSpec((_DH, 1), lambda k: (0, 0)),
            pl.BlockSpec((1, 1), lambda k: (0, 0)),
        ],
        out_specs=[
            pl.BlockSpec((1, _S), lambda k: (0, 0)),
            pl.BlockSpec((1, 1), lambda k: (0, 0)),
        ],
        out_shape=[
            jax.ShapeDtypeStruct((1, _S), jnp.float32),
            jax.ShapeDtypeStruct((1, 1), jnp.float32),
        ],
        scratch_shapes=[pltpu.VMEM((1, _DH), jnp.float32)],
    )(z3, wc, bc, wa, ba, wv, bv)


def kernel(g, input, inputs, W1, b1, W2, b2, Wc, bc, Wa, ba, Wv, bv):
    src2 = g[0].reshape(_EROWS, _LANE)
    dst2 = g[1].reshape(_EROWS, _LANE)

    degp = _sc_degrees(src2, dst2)
    degs = degp[0] + degp[1]
    no = (jnp.clip(degs[:_N], 1.0, None) ** -0.5).reshape(_N, 1)
    ni = (jnp.clip(degs[_NP:_NP + _N], 1.0, None) ** -0.5).reshape(_N, 1)

    y, cs, mm = _mm1(input.T, W1)
    h1 = _scale(y, cs, mm, no)
    agg1 = _agg128(h1, src2, dst2)

    w2p = jnp.zeros((_DH, 16), jnp.float32).at[:, :_S].set(W2)
    h2 = _mid(agg1, ni, b1.reshape(1, _DH), w2p, no)
    agg2 = _agg16(h2, src2, dst2)
    x2 = _post2(agg2, ni, b2.reshape(1, _S))

    zp = jnp.concatenate(
        [inputs, x2.reshape(-1),
         jnp.zeros((_ZPAD - _ZTOT,), jnp.float32)])
    z3 = zp.reshape(_KSTEPS, 1, _BK)
    actor, critic = _final(z3, Wc, bc.reshape(1, _DH), Wa,
                           ba.reshape(1, _S), Wv, bv.reshape(1, 1))
    return (actor.reshape(-1), critic.reshape(-1))


# speedup vs baseline: 10.9845x; 1.0583x over previous
import functools

import jax
import jax.numpy as jnp
from jax import lax
from jax.experimental import pallas as pl
from jax.experimental.pallas import tpu as pltpu
from jax.experimental.pallas import tpu_sc as plsc

_N = 10000
_E = 320000
_S = 6
_DIN = 2624
_DH = 128

_NC = 2
_NS = 16
_NW = _NC * _NS
_LANE = 128
_EROWS = _E // _LANE
_RPW = _EROWS // _NW
_EXTRA = _EROWS - _RPW * _NW

_mesh = plsc.VectorSubcoreMesh(core_axis_name="c", subcore_axis_name="s")


_K = 3
_SLOTS = _RPW // _K
_NP = 10240


def _row0(w):
    return w * _RPW + jnp.minimum(w, _EXTRA)


@functools.partial(
    pl.kernel,
    out_type=jax.ShapeDtypeStruct((_NC, 2 * _NP), jnp.float32),
    mesh=_mesh,
    compiler_params=pltpu.CompilerParams(use_tc_tiling_on_sc=False),
    scratch_types=[
        pltpu.VMEM((2, _K, _LANE), jnp.int32),
        pltpu.VMEM((2, _K, _LANE), jnp.int32),
        pltpu.VMEM((_LANE,), jnp.float32),
        pltpu.VMEM((1280,), jnp.float32),
        pltpu.VMEM_SHARED((2 * _NP,), jnp.float32),
        pltpu.SemaphoreType.DMA,
        pltpu.SemaphoreType.DMA,
    ],
)
def _sc_degrees(src_hbm, dst_hbm, out_hbm, srcv, dstv, ones_v, zero_v,
                deg_sh, sem_i0, sem_i1):
    c = lax.axis_index("c")
    s = lax.axis_index("s")
    w = s * _NC + c
    row0 = _row0(w)
    sems = (sem_i0, sem_i1)

    for j in range(_LANE // 16):
        ones_v[pl.ds(j * 16, 16)] = jnp.ones((16,), jnp.float32)

    def _zb(j, carry):
        zero_v[pl.ds(j * 16, 16)] = jnp.zeros((16,), jnp.float32)
        return carry

    lax.fori_loop(0, 1280 // 16, _zb, 0)
    pltpu.sync_copy(zero_v, deg_sh.at[pl.ds(s * 1280, 1280)])
    plsc.subcore_barrier()

    def _adjust(idxv, b):
        for j in range(_K):
            for jj in range(_LANE // 16):
                idxv[b, j, pl.ds(jj * 16, 16)] = (
                    idxv[b, j, pl.ds(jj * 16, 16)] + _NP)

    def _fetch(t, b):
        r = row0 + t * _K
        pltpu.async_copy(src_hbm.at[pl.ds(r, _K)], srcv.at[b], sems[b])
        pltpu.async_copy(dst_hbm.at[pl.ds(r, _K)], dstv.at[b], sems[b])

    def _drain(b):
        pltpu.make_async_copy(src_hbm.at[pl.ds(0, _K)], srcv.at[b],
                              sems[b]).wait()
        pltpu.make_async_copy(dst_hbm.at[pl.ds(0, _K)], dstv.at[b],
                              sems[b]).wait()
        _adjust(dstv, b)
        for j in range(_K):
            pltpu.sync_copy(ones_v, deg_sh.at[srcv.at[b, j]], add=True)
            pltpu.sync_copy(ones_v, deg_sh.at[dstv.at[b, j]], add=True)

    _fetch(0, 0)

    def _pair(u, carry):
        t0 = 2 * u
        _fetch(t0 + 1, 1)
        _drain(0)

        @pl.when(t0 + 2 < _SLOTS)
        def _():
            _fetch(t0 + 2, 0)

        _drain(1)
        return carry

    lax.fori_loop(0, _SLOTS // 2, _pair, 0)

    @pl.when(w < _EXTRA)
    def _():
        r = row0 + _RPW
        pltpu.sync_copy(src_hbm.at[pl.ds(r, 1)], srcv.at[0, pl.ds(0, 1)])
        pltpu.sync_copy(dst_hbm.at[pl.ds(r, 1)], dstv.at[0, pl.ds(0, 1)])
        for jj in range(_LANE // 16):
            dstv[0, 0, pl.ds(jj * 16, 16)] = (
                dstv[0, 0, pl.ds(jj * 16, 16)] + _NP)
        pltpu.sync_copy(ones_v, deg_sh.at[srcv.at[0, 0]], add=True)
        pltpu.sync_copy(ones_v, deg_sh.at[dstv.at[0, 0]], add=True)

    plsc.subcore_barrier()
    pltpu.sync_copy(deg_sh.at[pl.ds(s * 1280, 1280)],
                    out_hbm.at[c, pl.ds(s * 1280, 1280)])


def _make_agg(d, tc_tiling, k):
    slab = 640
    slots = _RPW // k
    assert slots % 3 == 0 and slots * k == _RPW
    zrows = k * _LANE

    def _zchunks(total):
        out = []
        while total:
            c = min(zrows, total)
            out.append(c)
            total -= c
        return out

    @functools.partial(
        pl.kernel,
        out_type=jax.ShapeDtypeStruct((_NC, _N, d), jnp.float32),
        mesh=_mesh,
        compiler_params=pltpu.CompilerParams(use_tc_tiling_on_sc=tc_tiling),
        scratch_types=[
            pltpu.VMEM((3, k, _LANE), jnp.int32),
            pltpu.VMEM((3, k, _LANE), jnp.int32),
            pltpu.VMEM((3, k * _LANE, d), jnp.float32),
            pltpu.VMEM_SHARED((_N, d), jnp.float32),
            pltpu.SemaphoreType.DMA,
            pltpu.SemaphoreType.DMA,
            pltpu.SemaphoreType.DMA,
            pltpu.SemaphoreType.DMA,
            pltpu.SemaphoreType.DMA,
            pltpu.SemaphoreType.DMA,
        ],
    )
    def _agg(h_hbm, src_hbm, dst_hbm, out_hbm, srcv, dstv, rows_v, acc_sh,
             sem_i0, sem_i1, sem_i2, sem_g0, sem_g1, sem_g2):
        c = lax.axis_index("c")
        s = lax.axis_index("s")
        w = s * _NC + c
        row0 = _row0(w)
        sems_i = (sem_i0, sem_i1, sem_i2)
        sems_g = (sem_g0, sem_g1, sem_g2)

        def _zb(i, carry):
            for j in range(d // 16):
                rows_v[0, i, pl.ds(j * 16, 16)] = jnp.zeros((16,),
                                                            jnp.float32)
            return carry

        lax.fori_loop(0, zrows, _zb, 0)
        base = s * slab
        off = 0
        for q in _zchunks(400):
            pltpu.sync_copy(rows_v.at[0, pl.ds(0, q)],
                            acc_sh.at[pl.ds(base + off, q)])
            off += q

        @pl.when(s < _NS - 1)
        def _():
            off2 = 400
            for q in _zchunks(slab - 400):
                pltpu.sync_copy(rows_v.at[0, pl.ds(0, q)],
                                acc_sh.at[pl.ds(base + off2, q)])
                off2 += q

        plsc.subcore_barrier()

        def _fetch(t, b):
            r = row0 + t * k
            for j in range(k):
                pltpu.async_copy(src_hbm.at[r + j], srcv.at[b, j], sems_i[b])
                pltpu.async_copy(dst_hbm.at[r + j], dstv.at[b, j], sems_i[b])
            for j in range(k):
                pltpu.make_async_copy(src_hbm.at[0], srcv.at[b, j],
                                      sems_i[b]).wait()
                pltpu.make_async_copy(dst_hbm.at[0], dstv.at[b, j],
                                      sems_i[b]).wait()
            for j in range(k):
                pltpu.async_copy(h_hbm.at[srcv.at[b, j]],
                                 rows_v.at[b, pl.ds(j * _LANE, _LANE)],
                                 sems_g[b])

        def _drain(b):
            for j in range(k):
                pltpu.make_async_copy(h_hbm.at[pl.ds(0, _LANE)],
                                      rows_v.at[b, pl.ds(j * _LANE, _LANE)],
                                      sems_g[b]).wait()
            for j in range(k):
                pltpu.sync_copy(rows_v.at[b, pl.ds(j * _LANE, _LANE)],
                                acc_sh.at[dstv.at[b, j]], add=True)

        _fetch(0, 0)
        _fetch(1, 1)

        def _trip(v, carry):
            for e in range(3):
                t = 3 * v + e

                @pl.when(t + 2 < slots)
                def _():
                    _fetch(t + 2, (e + 2) % 3)

                _drain(e)
            return carry

        lax.fori_loop(0, slots // 3, _trip, 0)

        @pl.when(w < _EXTRA)
        def _():
            r = row0 + _RPW
            pltpu.sync_copy(src_hbm.at[r], srcv.at[0, 0])
            pltpu.sync_copy(dst_hbm.at[r], dstv.at[0, 0])
            pltpu.async_copy(h_hbm.at[srcv.at[0, 0]],
                             rows_v.at[0, pl.ds(0, _LANE)], sem_g0)
            pltpu.make_async_copy(h_hbm.at[pl.ds(0, _LANE)],
                                  rows_v.at[0, pl.ds(0, _LANE)], sem_g0).wait()
            pltpu.sync_copy(rows_v.at[0, pl.ds(0, _LANE)],
                            acc_sh.at[dstv.at[0, 0]], add=True)

        plsc.subcore_barrier()
        pltpu.sync_copy(acc_sh.at[pl.ds(base, 400)],
                        out_hbm.at[c, pl.ds(base, 400)])

        @pl.when(s < _NS - 1)
        def _():
            pltpu.sync_copy(acc_sh.at[pl.ds(base + 400, 240)],
                            out_hbm.at[c, pl.ds(base + 400, 240)])

    return _agg


_agg128 = _make_agg(_DH, True, 1)
_agg16 = _make_agg(16, False, 2)


_BM = 1000
_BMX = 512
_MMG = 20


def _mm1_body(x_ref, w_ref, y_ref, cs_ref, mm_ref, acc):
    i = pl.program_id(0)
    last = pl.num_programs(0) - 1
    xb = x_ref[...]
    y_ref[...] = lax.dot_general(
        xb, w_ref[...], dimension_numbers=(((0,), (0,)), ((), ())),
        preferred_element_type=jnp.float32)

    @pl.when(i == 0)
    def _():
        acc[0] = jnp.inf
        acc[1] = -jnp.inf
        cs_ref[...] = jnp.sum(w_ref[...], axis=0, keepdims=True)

    @pl.when(i < last)
    def _():
        acc[0] = jnp.minimum(acc[0], jnp.min(xb))
        acc[1] = jnp.maximum(acc[1], jnp.max(xb))

    @pl.when(i == last)
    def _():
        col = lax.broadcasted_iota(jnp.int32, (_DIN, _BMX), 1) + i * _BMX
        valid = col < _N
        acc[0] = jnp.minimum(acc[0], jnp.min(jnp.where(valid, xb, jnp.inf)))
        acc[1] = jnp.maximum(acc[1], jnp.max(jnp.where(valid, xb, -jnp.inf)))
        mm_ref[0, 0] = acc[0]
        mm_ref[0, 1] = acc[1]


def _mm1(xt, w1):
    return pl.pallas_call(
        _mm1_body,
        grid=(_MMG,),
        in_specs=[
            pl.BlockSpec((_DIN, _BMX), lambda i: (0, i)),
            pl.BlockSpec((_DIN, _DH), lambda i: (0, 0)),
        ],
        out_specs=[
            pl.BlockSpec((_BMX, _DH), lambda i: (i, 0)),
            pl.BlockSpec((1, _DH), lambda i: (0, 0)),
            pl.BlockSpec((1, 2), lambda i: (0, 0), memory_space=pltpu.SMEM),
        ],
        out_shape=[
            jax.ShapeDtypeStruct((_N, _DH), jnp.float32),
            jax.ShapeDtypeStruct((1, _DH), jnp.float32),
            jax.ShapeDtypeStruct((1, 2), jnp.float32),
        ],
        scratch_shapes=[pltpu.SMEM((2,), jnp.float32)],
    )(xt, w1)


def _scale_body(y_ref, cs_ref, mm_ref, no_ref, h_ref):
    mn = mm_ref[0, 0]
    inv = 1.0 / (mm_ref[0, 1] - mn)
    h_ref[...] = (y_ref[...] - mn * cs_ref[...]) * inv * no_ref[...]


def _scale(y, cs, mm, no):
    return pl.pallas_call(
        _scale_body,
        grid=(_N // _BM,),
        in_specs=[
            pl.BlockSpec((_BM, _DH), lambda i: (i, 0)),
            pl.BlockSpec((1, _DH), lambda i: (0, 0)),
            pl.BlockSpec((1, 2), lambda i: (0, 0), memory_space=pltpu.SMEM),
            pl.BlockSpec((_BM, 1), lambda i: (i, 0)),
        ],
        out_specs=pl.BlockSpec((_BM, _DH), lambda i: (i, 0)),
        out_shape=jax.ShapeDtypeStruct((_N, _DH), jnp.float32),
    )(y, cs, mm, no)


def _mid_body(a_ref, ni_ref, b1_ref, w2_ref, no_ref, h2_ref):
    x1 = jax.nn.relu((a_ref[0] + a_ref[1]) * ni_ref[...] + b1_ref[...])
    h2_ref[...] = jnp.dot(
        x1, w2_ref[...], preferred_element_type=jnp.float32) * no_ref[...]


def _mid(agg1, ni, b1, w2p, no):
    return pl.pallas_call(
        _mid_body,
        grid=(_N // _BM,),
        in_specs=[
            pl.BlockSpec((_NC, _BM, _DH), lambda i: (0, i, 0)),
            pl.BlockSpec((_BM, 1), lambda i: (i, 0)),
            pl.BlockSpec((1, _DH), lambda i: (0, 0)),
            pl.BlockSpec((_DH, 16), lambda i: (0, 0)),
            pl.BlockSpec((_BM, 1), lambda i: (i, 0)),
        ],
        out_specs=pl.BlockSpec((_BM, 16), lambda i: (i, 0)),
        out_shape=jax.ShapeDtypeStruct((_N, 16), jnp.float32),
    )(agg1, ni, b1, w2p, no)


def _post2_body(a_ref, ni_ref, b2_ref, x2_ref):
    v = (a_ref[0] + a_ref[1]) * ni_ref[...]
    x2_ref[...] = v[:, :_S] + b2_ref[...]


def _post2(agg2, ni, b2):
    return pl.pallas_call(
        _post2_body,
        grid=(_N // _BM,),
        in_specs=[
            pl.BlockSpec((_NC, _BM, 16), lambda i: (0, i, 0)),
            pl.BlockSpec((_BM, 1), lambda i: (i, 0)),
            pl.BlockSpec((1, _S), lambda i: (0, 0)),
        ],
        out_specs=pl.BlockSpec((_BM, _S), lambda i: (i, 0)),
        out_shape=jax.ShapeDtypeStruct((_N, _S), jnp.float32),
    )(agg2, ni, b2)


_ZTOT = _N * _S + _N + _S + 1
_BK = 3584
_KSTEPS = 20
_ZPAD = _KSTEPS * _BK


def _final_body(z_ref, wc_ref, bc_ref, wa_ref, ba_ref, wv_ref, bv_ref,
                a_ref, c_ref, acc):
    k = pl.program_id(0)
    rows = lax.broadcasted_iota(jnp.int32, (_BK, _DH), 0) + k * _BK
    wc = jnp.where(rows < _ZTOT, wc_ref[...], 0.0)
    part = jnp.dot(z_ref[0], wc, preferred_element_type=jnp.float32)

    @pl.when(k == 0)
    def _():
        acc[...] = part

    @pl.when(k > 0)
    def _():
        acc[...] = acc[...] + part

    @pl.when(k == _KSTEPS - 1)
    def _():
        h = jax.nn.relu(acc[...] + bc_ref[...])
        a_ref[...] = jnp.dot(
            h, wa_ref[...], preferred_element_type=jnp.float32) + ba_ref[...]
        c_ref[...] = jnp.dot(
            h, wv_ref[...], preferred_element_type=jnp.float32) + bv_ref[...]


def _final(z3, wc, bc, wa, ba, wv, bv):
    return pl.pallas_call(
        _final_body,
        grid=(_KSTEPS,),
        in_specs=[
            pl.BlockSpec((1, 1, _BK), lambda k: (k, 0, 0)),
            pl.BlockSpec((_BK, _DH), lambda k: (k, 0)),
            pl.BlockSpec((1, _DH), lambda k: (0, 0)),
            pl.BlockSpec((_DH, _S), lambda k: (0, 0)),
            pl.BlockSpec((1, _S), lambda k: (0, 0)),
            pl.BlockSpec((_DH, 1), lambda k: (0, 0)),
            pl.BlockSpec((1, 1), lambda k: (0, 0)),
        ],
        out_specs=[
            pl.BlockSpec((1, _S), lambda k: (0, 0)),
            pl.BlockSpec((1, 1), lambda k: (0, 0)),
        ],
        out_shape=[
            jax.ShapeDtypeStruct((1, _S), jnp.float32),
            jax.ShapeDtypeStruct((1, 1), jnp.float32),
        ],
        scratch_shapes=[pltpu.VMEM((1, _DH), jnp.float32)],
    )(z3, wc, bc, wa, ba, wv, bv)


def kernel(g, input, inputs, W1, b1, W2, b2, Wc, bc, Wa, ba, Wv, bv):
    src2 = g[0].reshape(_EROWS, _LANE)
    dst2 = g[1].reshape(_EROWS, _LANE)

    degp = _sc_degrees(src2, dst2)
    degs = degp[0] + degp[1]
    no = (jnp.clip(degs[:_N], 1.0, None) ** -0.5).reshape(_N, 1)
    ni = (jnp.clip(degs[_NP:_NP + _N], 1.0, None) ** -0.5).reshape(_N, 1)

    y, cs, mm = _mm1(input.T, W1)
    h1 = _scale(y, cs, mm, no)
    agg1 = _agg128(h1, src2, dst2)

    w2p = jnp.zeros((_DH, 16), jnp.float32).at[:, :_S].set(W2)
    h2 = _mid(agg1, ni, b1.reshape(1, _DH), w2p, no)
    agg2 = _agg16(h2, src2, dst2)
    x2 = _post2(agg2, ni, b2.reshape(1, _S))

    zp = jnp.concatenate(
        [inputs, x2.reshape(-1),
         jnp.zeros((_ZPAD - _ZTOT,), jnp.float32)])
    z3 = zp.reshape(_KSTEPS, 1, _BK)
    actor, critic = _final(z3, Wc, bc.reshape(1, _DH), Wa,
                           ba.reshape(1, _S), Wv, bv.reshape(1, 1))
    return (actor.reshape(-1), critic.reshape(-1))


# speedup vs baseline: 11.6307x; 1.0588x over previous
import functools

import jax
import jax.numpy as jnp
from jax import lax
from jax.experimental import pallas as pl
from jax.experimental.pallas import tpu as pltpu
from jax.experimental.pallas import tpu_sc as plsc

_N = 10000
_E = 320000
_S = 6
_DIN = 2624
_DH = 128

_NC = 2
_NS = 16
_NW = _NC * _NS
_LANE = 128
_EROWS = _E // _LANE
_RPW = _EROWS // _NW
_EXTRA = _EROWS - _RPW * _NW

_mesh = plsc.VectorSubcoreMesh(core_axis_name="c", subcore_axis_name="s")


_K = 3
_SLOTS = _RPW // _K
_NP = 10240


def _row0(w):
    return w * _RPW + jnp.minimum(w, _EXTRA)


@functools.partial(
    pl.kernel,
    out_type=jax.ShapeDtypeStruct((_NC, 2 * _NP), jnp.float32),
    mesh=_mesh,
    compiler_params=pltpu.CompilerParams(use_tc_tiling_on_sc=False),
    scratch_types=[
        pltpu.VMEM((2, _K, _LANE), jnp.int32),
        pltpu.VMEM((2, _K, _LANE), jnp.int32),
        pltpu.VMEM((_LANE,), jnp.float32),
        pltpu.VMEM((1280,), jnp.float32),
        pltpu.VMEM_SHARED((2 * _NP,), jnp.float32),
        pltpu.SemaphoreType.DMA,
        pltpu.SemaphoreType.DMA,
    ],
)
def _sc_degrees(src_hbm, dst_hbm, out_hbm, srcv, dstv, ones_v, zero_v,
                deg_sh, sem_i0, sem_i1):
    c = lax.axis_index("c")
    s = lax.axis_index("s")
    w = s * _NC + c
    row0 = _row0(w)
    sems = (sem_i0, sem_i1)

    for j in range(_LANE // 16):
        ones_v[pl.ds(j * 16, 16)] = jnp.ones((16,), jnp.float32)

    def _zb(j, carry):
        zero_v[pl.ds(j * 16, 16)] = jnp.zeros((16,), jnp.float32)
        return carry

    lax.fori_loop(0, 1280 // 16, _zb, 0)
    pltpu.sync_copy(zero_v, deg_sh.at[pl.ds(s * 1280, 1280)])
    plsc.subcore_barrier()

    def _adjust(idxv, b):
        for j in range(_K):
            for jj in range(_LANE // 16):
                idxv[b, j, pl.ds(jj * 16, 16)] = (
                    idxv[b, j, pl.ds(jj * 16, 16)] + _NP)

    def _fetch(t, b):
        r = row0 + t * _K
        pltpu.async_copy(src_hbm.at[pl.ds(r, _K)], srcv.at[b], sems[b])
        pltpu.async_copy(dst_hbm.at[pl.ds(r, _K)], dstv.at[b], sems[b])

    def _drain(b):
        pltpu.make_async_copy(src_hbm.at[pl.ds(0, _K)], srcv.at[b],
                              sems[b]).wait()
        pltpu.make_async_copy(dst_hbm.at[pl.ds(0, _K)], dstv.at[b],
                              sems[b]).wait()
        _adjust(dstv, b)
        for j in range(_K):
            pltpu.sync_copy(ones_v, deg_sh.at[srcv.at[b, j]], add=True)
            pltpu.sync_copy(ones_v, deg_sh.at[dstv.at[b, j]], add=True)

    _fetch(0, 0)

    def _pair(u, carry):
        t0 = 2 * u
        _fetch(t0 + 1, 1)
        _drain(0)

        @pl.when(t0 + 2 < _SLOTS)
        def _():
            _fetch(t0 + 2, 0)

        _drain(1)
        return carry

    lax.fori_loop(0, _SLOTS // 2, _pair, 0)

    @pl.when(w < _EXTRA)
    def _():
        r = row0 + _RPW
        pltpu.sync_copy(src_hbm.at[pl.ds(r, 1)], srcv.at[0, pl.ds(0, 1)])
        pltpu.sync_copy(dst_hbm.at[pl.ds(r, 1)], dstv.at[0, pl.ds(0, 1)])
        for jj in range(_LANE // 16):
            dstv[0, 0, pl.ds(jj * 16, 16)] = (
                dstv[0, 0, pl.ds(jj * 16, 16)] + _NP)
        pltpu.sync_copy(ones_v, deg_sh.at[srcv.at[0, 0]], add=True)
        pltpu.sync_copy(ones_v, deg_sh.at[dstv.at[0, 0]], add=True)

    plsc.subcore_barrier()
    pltpu.sync_copy(deg_sh.at[pl.ds(s * 1280, 1280)],
                    out_hbm.at[c, pl.ds(s * 1280, 1280)])


def _make_agg(d, tc_tiling, k):
    slab = 640
    slots = _RPW // k
    assert slots % 3 == 0 and slots * k == _RPW
    zrows = k * _LANE

    def _zchunks(total):
        out = []
        while total:
            c = min(zrows, total)
            out.append(c)
            total -= c
        return out

    @functools.partial(
        pl.kernel,
        out_type=jax.ShapeDtypeStruct((_NC, _N, d), jnp.float32),
        mesh=_mesh,
        compiler_params=pltpu.CompilerParams(use_tc_tiling_on_sc=tc_tiling),
        scratch_types=[
            pltpu.VMEM((3, k, _LANE), jnp.int32),
            pltpu.VMEM((3, k, _LANE), jnp.int32),
            pltpu.VMEM((3, k * _LANE, d), jnp.float32),
            pltpu.VMEM_SHARED((_N, d), jnp.float32),
            pltpu.SemaphoreType.DMA,
            pltpu.SemaphoreType.DMA,
            pltpu.SemaphoreType.DMA,
            pltpu.SemaphoreType.DMA,
            pltpu.SemaphoreType.DMA,
            pltpu.SemaphoreType.DMA,
        ],
    )
    def _agg(h_hbm, src_hbm, dst_hbm, out_hbm, srcv, dstv, rows_v, acc_sh,
             sem_i0, sem_i1, sem_i2, sem_g0, sem_g1, sem_g2):
        c = lax.axis_index("c")
        s = lax.axis_index("s")
        w = s * _NC + c
        row0 = _row0(w)
        sems_i = (sem_i0, sem_i1, sem_i2)
        sems_g = (sem_g0, sem_g1, sem_g2)

        def _zb(i, carry):
            for j in range(d // 16):
                rows_v[0, i, pl.ds(j * 16, 16)] = jnp.zeros((16,),
                                                            jnp.float32)
            return carry

        lax.fori_loop(0, zrows, _zb, 0)
        base = s * slab
        off = 0
        for q in _zchunks(400):
            pltpu.sync_copy(rows_v.at[0, pl.ds(0, q)],
                            acc_sh.at[pl.ds(base + off, q)])
            off += q

        @pl.when(s < _NS - 1)
        def _():
            off2 = 400
            for q in _zchunks(slab - 400):
                pltpu.sync_copy(rows_v.at[0, pl.ds(0, q)],
                                acc_sh.at[pl.ds(base + off2, q)])
                off2 += q

        plsc.subcore_barrier()

        def _fetch(t, b):
            r = row0 + t * k
            for j in range(k):
                pltpu.async_copy(src_hbm.at[r + j], srcv.at[b, j], sems_i[b])
                pltpu.async_copy(dst_hbm.at[r + j], dstv.at[b, j], sems_i[b])
            for j in range(k):
                pltpu.make_async_copy(src_hbm.at[0], srcv.at[b, j],
                                      sems_i[b]).wait()
                pltpu.make_async_copy(dst_hbm.at[0], dstv.at[b, j],
                                      sems_i[b]).wait()
            for j in range(k):
                pltpu.async_copy(h_hbm.at[srcv.at[b, j]],
                                 rows_v.at[b, pl.ds(j * _LANE, _LANE)],
                                 sems_g[b])

        def _drain(b):
            for j in range(k):
                pltpu.make_async_copy(h_hbm.at[pl.ds(0, _LANE)],
                                      rows_v.at[b, pl.ds(j * _LANE, _LANE)],
                                      sems_g[b]).wait()
            for j in range(k):
                pltpu.sync_copy(rows_v.at[b, pl.ds(j * _LANE, _LANE)],
                                acc_sh.at[dstv.at[b, j]], add=True)

        _fetch(0, 0)
        _fetch(1, 1)

        def _trip(v, carry):
            for e in range(3):
                t = 3 * v + e

                @pl.when(t + 2 < slots)
                def _():
                    _fetch(t + 2, (e + 2) % 3)

                _drain(e)
            return carry

        lax.fori_loop(0, slots // 3, _trip, 0)

        @pl.when(w < _EXTRA)
        def _():
            r = row0 + _RPW
            pltpu.sync_copy(src_hbm.at[r], srcv.at[0, 0])
            pltpu.sync_copy(dst_hbm.at[r], dstv.at[0, 0])
            pltpu.async_copy(h_hbm.at[srcv.at[0, 0]],
                             rows_v.at[0, pl.ds(0, _LANE)], sem_g0)
            pltpu.make_async_copy(h_hbm.at[pl.ds(0, _LANE)],
                                  rows_v.at[0, pl.ds(0, _LANE)], sem_g0).wait()
            pltpu.sync_copy(rows_v.at[0, pl.ds(0, _LANE)],
                            acc_sh.at[dstv.at[0, 0]], add=True)

        plsc.subcore_barrier()
        pltpu.sync_copy(acc_sh.at[pl.ds(base, 400)],
                        out_hbm.at[c, pl.ds(base, 400)])

        @pl.when(s < _NS - 1)
        def _():
            pltpu.sync_copy(acc_sh.at[pl.ds(base + 400, 240)],
                            out_hbm.at[c, pl.ds(base + 400, 240)])

    return _agg


_agg128 = _make_agg(_DH, True, 1)
_agg16 = _make_agg(16, False, 13)


_BM = 1000
_BMX = 512
_MMG = 20


def _mm1_body(x_ref, w_ref, y_ref, cs_ref, mm_ref, acc):
    i = pl.program_id(0)
    last = pl.num_programs(0) - 1
    xb = x_ref[...]
    y_ref[...] = lax.dot_general(
        xb, w_ref[...], dimension_numbers=(((0,), (0,)), ((), ())),
        preferred_element_type=jnp.float32)

    @pl.when(i == 0)
    def _():
        acc[0] = jnp.inf
        acc[1] = -jnp.inf
        cs_ref[...] = jnp.sum(w_ref[...], axis=0, keepdims=True)

    @pl.when(i < last)
    def _():
        acc[0] = jnp.minimum(acc[0], jnp.min(xb))
        acc[1] = jnp.maximum(acc[1], jnp.max(xb))

    @pl.when(i == last)
    def _():
        col = lax.broadcasted_iota(jnp.int32, (_DIN, _BMX), 1) + i * _BMX
        valid = col < _N
        acc[0] = jnp.minimum(acc[0], jnp.min(jnp.where(valid, xb, jnp.inf)))
        acc[1] = jnp.maximum(acc[1], jnp.max(jnp.where(valid, xb, -jnp.inf)))
        mm_ref[0, 0] = acc[0]
        mm_ref[0, 1] = acc[1]


def _mm1(xt, w1):
    return pl.pallas_call(
        _mm1_body,
        grid=(_MMG,),
        in_specs=[
            pl.BlockSpec((_DIN, _BMX), lambda i: (0, i)),
            pl.BlockSpec((_DIN, _DH), lambda i: (0, 0)),
        ],
        out_specs=[
            pl.BlockSpec((_BMX, _DH), lambda i: (i, 0)),
            pl.BlockSpec((1, _DH), lambda i: (0, 0)),
            pl.BlockSpec((1, 2), lambda i: (0, 0), memory_space=pltpu.SMEM),
        ],
        out_shape=[
            jax.ShapeDtypeStruct((_N, _DH), jnp.float32),
            jax.ShapeDtypeStruct((1, _DH), jnp.float32),
            jax.ShapeDtypeStruct((1, 2), jnp.float32),
        ],
        scratch_shapes=[pltpu.SMEM((2,), jnp.float32)],
    )(xt, w1)


def _scale_body(y_ref, cs_ref, mm_ref, no_ref, h_ref):
    mn = mm_ref[0, 0]
    inv = 1.0 / (mm_ref[0, 1] - mn)
    h_ref[...] = (y_ref[...] - mn * cs_ref[...]) * inv * no_ref[...]


def _scale(y, cs, mm, no):
    return pl.pallas_call(
        _scale_body,
        grid=(_N // _BM,),
        in_specs=[
            pl.BlockSpec((_BM, _DH), lambda i: (i, 0)),
            pl.BlockSpec((1, _DH), lambda i: (0, 0)),
            pl.BlockSpec((1, 2), lambda i: (0, 0), memory_space=pltpu.SMEM),
            pl.BlockSpec((_BM, 1), lambda i: (i, 0)),
        ],
        out_specs=pl.BlockSpec((_BM, _DH), lambda i: (i, 0)),
        out_shape=jax.ShapeDtypeStruct((_N, _DH), jnp.float32),
    )(y, cs, mm, no)


def _mid_body(a_ref, ni_ref, b1_ref, w2_ref, no_ref, h2_ref):
    x1 = jax.nn.relu((a_ref[0] + a_ref[1]) * ni_ref[...] + b1_ref[...])
    h2_ref[...] = jnp.dot(
        x1, w2_ref[...], preferred_element_type=jnp.float32) * no_ref[...]


def _mid(agg1, ni, b1, w2p, no):
    return pl.pallas_call(
        _mid_body,
        grid=(_N // _BM,),
        in_specs=[
            pl.BlockSpec((_NC, _BM, _DH), lambda i: (0, i, 0)),
            pl.BlockSpec((_BM, 1), lambda i: (i, 0)),
            pl.BlockSpec((1, _DH), lambda i: (0, 0)),
            pl.BlockSpec((_DH, 16), lambda i: (0, 0)),
            pl.BlockSpec((_BM, 1), lambda i: (i, 0)),
        ],
        out_specs=pl.BlockSpec((_BM, 16), lambda i: (i, 0)),
        out_shape=jax.ShapeDtypeStruct((_N, 16), jnp.float32),
    )(agg1, ni, b1, w2p, no)


_ZTOT = _N * _S + _N + _S + 1
_BK = 7168
_KSTEPS = 10
_ZPAD = _KSTEPS * _BK


def _final_body(z_ref, wc_ref, bc_ref, wa_ref, ba_ref, wv_ref, bv_ref,
                a_ref, c_ref, acc):
    k = pl.program_id(0)
    rows = lax.broadcasted_iota(jnp.int32, (_BK, _DH), 0) + k * _BK
    wc = jnp.where(rows < _ZTOT, wc_ref[...], 0.0)
    part = jnp.dot(z_ref[0], wc, preferred_element_type=jnp.float32)

    @pl.when(k == 0)
    def _():
        acc[...] = part

    @pl.when(k > 0)
    def _():
        acc[...] = acc[...] + part

    @pl.when(k == _KSTEPS - 1)
    def _():
        h = jax.nn.relu(acc[...] + bc_ref[...])
        a_ref[...] = jnp.dot(
            h, wa_ref[...], preferred_element_type=jnp.float32) + ba_ref[...]
        c_ref[...] = jnp.dot(
            h, wv_ref[...], preferred_element_type=jnp.float32) + bv_ref[...]


def _final(z3, wc, bc, wa, ba, wv, bv):
    return pl.pallas_call(
        _final_body,
        grid=(_KSTEPS,),
        in_specs=[
            pl.BlockSpec((1, 1, _BK), lambda k: (k, 0, 0)),
            pl.BlockSpec((_BK, _DH), lambda k: (k, 0)),
            pl.BlockSpec((1, _DH), lambda k: (0, 0)),
            pl.BlockSpec((_DH, _S), lambda k: (0, 0)),
            pl.BlockSpec((1, _S), lambda k: (0, 0)),
            pl.BlockSpec((_DH, 1), lambda k: (0, 0)),
            pl.BlockSpec((1, 1), lambda k: (0, 0)),
        ],
        out_specs=[
            pl.BlockSpec((1, _S), lambda k: (0, 0)),
            pl.BlockSpec((1, 1), lambda k: (0, 0)),
        ],
        out_shape=[
            jax.ShapeDtypeStruct((1, _S), jnp.float32),
            jax.ShapeDtypeStruct((1, 1), jnp.float32),
        ],
        scratch_shapes=[pltpu.VMEM((1, _DH), jnp.float32)],
    )(z3, wc, bc, wa, ba, wv, bv)


def kernel(g, input, inputs, W1, b1, W2, b2, Wc, bc, Wa, ba, Wv, bv):
    src2 = g[0].reshape(_EROWS, _LANE)
    dst2 = g[1].reshape(_EROWS, _LANE)

    degp = _sc_degrees(src2, dst2)
    degs = degp[0] + degp[1]
    no = (jnp.clip(degs[:_N], 1.0, None) ** -0.5).reshape(_N, 1)
    ni = (jnp.clip(degs[_NP:_NP + _N], 1.0, None) ** -0.5).reshape(_N, 1)

    y, cs, mm = _mm1(input.T, W1)
    h1 = _scale(y, cs, mm, no)
    agg1 = _agg128(h1, src2, dst2)

    w2p = jnp.zeros((_DH, 16), jnp.float32).at[:, :_S].set(W2)
    h2 = _mid(agg1, ni, b1.reshape(1, _DH), w2p, no)
    agg2 = _agg16(h2, src2, dst2)
    x2 = (agg2[0] + agg2[1])[:, :_S] * ni + b2.reshape(1, _S)

    zp = jnp.concatenate(
        [inputs, x2.reshape(-1),
         jnp.zeros((_ZPAD - _ZTOT,), jnp.float32)])
    z3 = zp.reshape(_KSTEPS, 1, _BK)
    actor, critic = _final(z3, Wc, bc.reshape(1, _DH), Wa,
                           ba.reshape(1, _S), Wv, bv.reshape(1, 1))
    return (actor.reshape(-1), critic.reshape(-1))
